# baseline (speedup 1.0000x reference)
import functools

import jax
import jax.numpy as jnp
from jax import lax
from jax.experimental import pallas as pl
from jax.experimental.pallas import tpu as pltpu
from jax.experimental.pallas import tpu_sc as plsc

N_NODES = 10000
IN_CH = 128
HID = 16
NUM_CLASSES = 40

NC = 2
NS = 16
NW = NC * NS
CHUNK = 125
NCHUNK = 80
ACC_ROWS = 10240
RPT = ACC_ROWS // NS

BR = 1000
GRID = N_NODES // BR


def _scatter_add_call(feat, src3, dst3, zeros, F):
    mesh = plsc.VectorSubcoreMesh(core_axis_name="c", subcore_axis_name="s")

    @functools.partial(
        pl.kernel,
        out_type=(
            jax.ShapeDtypeStruct((ACC_ROWS, F), jnp.float32),
            jax.ShapeDtypeStruct((ACC_ROWS, F), jnp.float32),
        ),
        mesh=mesh,
        scratch_types=[
            pltpu.VMEM((NCHUNK, CHUNK), jnp.int32),
            pltpu.VMEM((NCHUNK, CHUNK), jnp.int32),
            pltpu.VMEM((8, CHUNK, F), jnp.float32),
            pltpu.VMEM_SHARED((ACC_ROWS, F), jnp.float32),
            pltpu.VMEM_SHARED((N_NODES, F), jnp.float32),
            pltpu.SemaphoreType.DMA,
            pltpu.SemaphoreType.DMA,
            pltpu.SemaphoreType.DMA,
        ],
        compiler_params=pltpu.CompilerParams(use_tc_tiling_on_sc=False),
    )
    def k(feat_hbm, src_hbm, dst_hbm, zero_hbm, out0, out1,
          srcv, dstv, rows, acc, tbl, sem, sem2, sems):
        cid = lax.axis_index("c")
        sid = lax.axis_index("s")
        wid = sid * NC + cid
        tpt = N_NODES // NS
        z = pltpu.async_copy(zero_hbm, acc.at[pl.ds(sid * RPT, RPT)], sem2)
        t = pltpu.async_copy(feat_hbm.at[pl.ds(sid * tpt, tpt)],
                             tbl.at[pl.ds(sid * tpt, tpt)], sem2)
        s = pltpu.async_copy(src_hbm.at[wid], srcv, sem2)
        d = pltpu.async_copy(dst_hbm.at[wid], dstv, sem2)
        z.wait(); t.wait(); s.wait(); d.wait()
        plsc.subcore_barrier()

        for j0 in range(4):
            pltpu.async_copy(tbl.at[srcv.at[j0]], rows.at[j0], sem)

        def body(j, carry):
            @pl.when(j >= 4)
            def _():
                pltpu.make_async_copy(rows.at[lax.rem(j, 8)],
                                      acc.at[dstv.at[j - 4]], sems).wait()

            @pl.when(j + 4 < NCHUNK)
            def _():
                jn = j + 4
                pltpu.async_copy(tbl.at[srcv.at[jn]],
                                 rows.at[lax.rem(jn, 8)], sem)

            cur = lax.rem(j, 8)
            pltpu.make_async_copy(tbl.at[srcv.at[j]], rows.at[cur],
                                  sem).wait()
            pltpu.async_copy(rows.at[cur], acc.at[dstv.at[j]], sems,
                             add=True)
            return carry

        lax.fori_loop(0, NCHUNK, body, 0)
        for b in range(4):
            pltpu.make_async_copy(rows.at[b], acc.at[dstv.at[0]],
                                  sems).wait()
        plsc.subcore_barrier()

        @pl.when(cid == 0)
        def _():
            pltpu.sync_copy(acc.at[pl.ds(sid * RPT, RPT)],
                            out0.at[pl.ds(sid * RPT, RPT)])

        @pl.when(cid == 1)
        def _():
            pltpu.sync_copy(acc.at[pl.ds(sid * RPT, RPT)],
                            out1.at[pl.ds(sid * RPT, RPT)])

    return k(feat, src3, dst3, zeros)


def _proj_call(x, w):
    def body(x_ref, w_ref, o_ref):
        o_ref[...] = jnp.dot(x_ref[...], w_ref[...],
                             preferred_element_type=jnp.float32)

    return pl.pallas_call(
        body,
        grid=(5,),
        in_specs=[
            pl.BlockSpec((2000, IN_CH), lambda i: (i, 0)),
            pl.BlockSpec((IN_CH, 2 * HID), lambda i: (0, 0)),
        ],
        out_specs=pl.BlockSpec((2000, 2 * HID), lambda i: (i, 0)),
        out_shape=jax.ShapeDtypeStruct((N_NODES, 2 * HID), jnp.float32),
    )(x, w)


def _mlp1_call(p, a0, a1, b1, W2, b2):
    def body(p_ref, a0_ref, a1_ref, b1_ref, W2_ref, b2_ref, o_ref):
        t = p_ref[...] + a0_ref[...] + a1_ref[...] + b1_ref[...]
        t = jnp.maximum(t, 0.0)
        h = jnp.dot(t, W2_ref[...], preferred_element_type=jnp.float32)
        o_ref[...] = jnp.maximum(h + b2_ref[...], 0.0)

    return pl.pallas_call(
        body,
        grid=(2,),
        in_specs=[
            pl.BlockSpec((N_NODES // 2, 2 * HID), lambda i: (i, 0)),
            pl.BlockSpec((N_NODES // 2, 2 * HID), lambda i: (i, 0)),
            pl.BlockSpec((N_NODES // 2, 2 * HID), lambda i: (i, 0)),
            pl.BlockSpec((1, 2 * HID), lambda i: (0, 0)),
            pl.BlockSpec((2 * HID, HID), lambda i: (0, 0)),
            pl.BlockSpec((1, HID), lambda i: (0, 0)),
        ],
        out_specs=pl.BlockSpec((N_NODES // 2, HID), lambda i: (i, 0)),
        out_shape=jax.ShapeDtypeStruct((N_NODES, HID), jnp.float32),
    )(p, a0, a1, b1, W2, b2)


def _mlp2_call(h1, a0, a1, W1, b1, W2, b2, fcW, fcb):
    def body(h_ref, a0_ref, a1_ref, W1_ref, b1_ref, W2_ref, b2_ref,
             fcW_ref, fcb_ref, o_ref):
        t = h_ref[...] + a0_ref[...] + a1_ref[...]
        u = jnp.dot(t, W1_ref[...], preferred_element_type=jnp.float32)
        u = jnp.maximum(u + b1_ref[...], 0.0)
        v = jnp.dot(u, W2_ref[...], preferred_element_type=jnp.float32)
        v = jnp.maximum(v + b2_ref[...], 0.0)
        o_ref[...] = jnp.dot(v, fcW_ref[...],
                             preferred_element_type=jnp.float32) + fcb_ref[...]

    return pl.pallas_call(
        body,
        grid=(2,),
        in_specs=[
            pl.BlockSpec((N_NODES // 2, HID), lambda i: (i, 0)),
            pl.BlockSpec((N_NODES // 2, HID), lambda i: (i, 0)),
            pl.BlockSpec((N_NODES // 2, HID), lambda i: (i, 0)),
            pl.BlockSpec((HID, 2 * HID), lambda i: (0, 0)),
            pl.BlockSpec((1, 2 * HID), lambda i: (0, 0)),
            pl.BlockSpec((2 * HID, HID), lambda i: (0, 0)),
            pl.BlockSpec((1, HID), lambda i: (0, 0)),
            pl.BlockSpec((HID, NUM_CLASSES), lambda i: (0, 0)),
            pl.BlockSpec((1, NUM_CLASSES), lambda i: (0, 0)),
        ],
        out_specs=pl.BlockSpec((N_NODES // 2, NUM_CLASSES), lambda i: (i, 0)),
        out_shape=jax.ShapeDtypeStruct((N_NODES, NUM_CLASSES), jnp.float32),
    )(h1, a0, a1, W1, b1, W2, b2, fcW, fcb)


def kernel(x, edge_index, c1W1, c1b1, c1W2, c1b2,
           c2W1, c2b1, c2W2, c2b2, fcW, fcb):
    ei = edge_index.astype(jnp.int32)
    src3 = ei[0].reshape(NW, NCHUNK, CHUNK)
    dst3 = ei[1].reshape(NW, NCHUNK, CHUNK)
    zeros32 = jnp.zeros((RPT, 2 * HID), jnp.float32)
    zeros16 = jnp.zeros((RPT, HID), jnp.float32)

    p = _proj_call(x, c1W1)
    a0, a1 = _scatter_add_call(p, src3, dst3, zeros32, 2 * HID)
    h1 = _mlp1_call(p, a0, a1, c1b1.reshape(1, -1), c1W2,
                    c1b2.reshape(1, -1))
    g0, g1 = _scatter_add_call(h1, src3, dst3, zeros16, HID)
    logits = _mlp2_call(h1, g0, g1, c2W1, c2b1.reshape(1, -1), c2W2,
                        c2b2.reshape(1, -1), fcW, fcb.reshape(1, -1))
    return logits


# speedup vs baseline: 1.0013x; 1.0013x over previous
import functools

import jax
import jax.numpy as jnp
from jax import lax
from jax.experimental import pallas as pl
from jax.experimental.pallas import tpu as pltpu
from jax.experimental.pallas import tpu_sc as plsc

N_NODES = 10000
IN_CH = 128
HID = 16
NUM_CLASSES = 40

NC = 2
NS = 16
NW = NC * NS
CHUNK = 100
NCHUNK = 100
ACC_ROWS = 10240
RPT = ACC_ROWS // NS


def _scatter_add_call(feat, src3, dst3, zeros, F):
    mesh = plsc.VectorSubcoreMesh(core_axis_name="c", subcore_axis_name="s")

    @functools.partial(
        pl.kernel,
        out_type=(
            jax.ShapeDtypeStruct((ACC_ROWS, F), jnp.float32),
            jax.ShapeDtypeStruct((ACC_ROWS, F), jnp.float32),
        ),
        mesh=mesh,
        scratch_types=[
            pltpu.VMEM((NCHUNK, CHUNK), jnp.int32),
            pltpu.VMEM((NCHUNK, CHUNK), jnp.int32),
            pltpu.VMEM((8, CHUNK, F), jnp.float32),
            pltpu.VMEM_SHARED((ACC_ROWS, F), jnp.float32),
            pltpu.VMEM_SHARED((N_NODES, F), jnp.float32),
            pltpu.SemaphoreType.DMA,
            pltpu.SemaphoreType.DMA,
            pltpu.SemaphoreType.DMA,
        ],
        compiler_params=pltpu.CompilerParams(use_tc_tiling_on_sc=False),
    )
    def k(feat_hbm, src_hbm, dst_hbm, zero_hbm, out0, out1,
          srcv, dstv, rows, acc, tbl, sem, sem2, sems):
        cid = lax.axis_index("c")
        sid = lax.axis_index("s")
        wid = sid * NC + cid
        tpt = N_NODES // NS
        z = pltpu.async_copy(zero_hbm, acc.at[pl.ds(sid * RPT, RPT)], sem2)
        t = pltpu.async_copy(feat_hbm.at[pl.ds(sid * tpt, tpt)],
                             tbl.at[pl.ds(sid * tpt, tpt)], sem2)
        s = pltpu.async_copy(src_hbm.at[wid], srcv, sem2)
        d = pltpu.async_copy(dst_hbm.at[wid], dstv, sem2)
        z.wait(); t.wait(); s.wait(); d.wait()
        plsc.subcore_barrier()

        for j0 in range(4):
            pltpu.async_copy(tbl.at[srcv.at[j0]], rows.at[j0], sem)

        def body(j, carry):
            @pl.when(j >= 4)
            def _():
                pltpu.make_async_copy(rows.at[lax.rem(j, 8)],
                                      acc.at[dstv.at[j - 4]], sems).wait()

            @pl.when(j + 4 < NCHUNK)
            def _():
                jn = j + 4
                pltpu.async_copy(tbl.at[srcv.at[jn]],
                                 rows.at[lax.rem(jn, 8)], sem)

            cur = lax.rem(j, 8)
            pltpu.make_async_copy(tbl.at[srcv.at[j]], rows.at[cur],
                                  sem).wait()
            pltpu.async_copy(rows.at[cur], acc.at[dstv.at[j]], sems,
                             add=True)
            return carry

        lax.fori_loop(0, NCHUNK, body, 0)
        for b in range(4):
            pltpu.make_async_copy(rows.at[b], acc.at[dstv.at[0]],
                                  sems).wait()
        plsc.subcore_barrier()

        @pl.when(cid == 0)
        def _():
            pltpu.sync_copy(acc.at[pl.ds(sid * RPT, RPT)],
                            out0.at[pl.ds(sid * RPT, RPT)])

        @pl.when(cid == 1)
        def _():
            pltpu.sync_copy(acc.at[pl.ds(sid * RPT, RPT)],
                            out1.at[pl.ds(sid * RPT, RPT)])

    return k(feat, src3, dst3, zeros)


def _proj_call(x, w):
    def body(x_ref, w_ref, o_ref):
        o_ref[...] = jnp.dot(x_ref[...], w_ref[...],
                             preferred_element_type=jnp.float32)

    return pl.pallas_call(
        body,
        grid=(5,),
        in_specs=[
            pl.BlockSpec((2000, IN_CH), lambda i: (i, 0)),
            pl.BlockSpec((IN_CH, 2 * HID), lambda i: (0, 0)),
        ],
        out_specs=pl.BlockSpec((2000, 2 * HID), lambda i: (i, 0)),
        out_shape=jax.ShapeDtypeStruct((N_NODES, 2 * HID), jnp.float32),
    )(x, w)


def _mlp1_call(p, a0, a1, b1, W2, b2):
    def body(p_ref, a0_ref, a1_ref, b1_ref, W2_ref, b2_ref, o_ref):
        t = p_ref[...] + a0_ref[...] + a1_ref[...] + b1_ref[...]
        t = jnp.maximum(t, 0.0)
        h = jnp.dot(t, W2_ref[...], preferred_element_type=jnp.float32)
        o_ref[...] = jnp.maximum(h + b2_ref[...], 0.0)

    return pl.pallas_call(
        body,
        grid=(2,),
        in_specs=[
            pl.BlockSpec((N_NODES // 2, 2 * HID), lambda i: (i, 0)),
            pl.BlockSpec((N_NODES // 2, 2 * HID), lambda i: (i, 0)),
            pl.BlockSpec((N_NODES // 2, 2 * HID), lambda i: (i, 0)),
            pl.BlockSpec((1, 2 * HID), lambda i: (0, 0)),
            pl.BlockSpec((2 * HID, HID), lambda i: (0, 0)),
            pl.BlockSpec((1, HID), lambda i: (0, 0)),
        ],
        out_specs=pl.BlockSpec((N_NODES // 2, HID), lambda i: (i, 0)),
        out_shape=jax.ShapeDtypeStruct((N_NODES, HID), jnp.float32),
    )(p, a0, a1, b1, W2, b2)


def _mlp2_call(h1, a0, a1, W1, b1, W2, b2, fcW, fcb):
    def body(h_ref, a0_ref, a1_ref, W1_ref, b1_ref, W2_ref, b2_ref,
             fcW_ref, fcb_ref, o_ref):
        t = h_ref[...] + a0_ref[...] + a1_ref[...]
        u = jnp.dot(t, W1_ref[...], preferred_element_type=jnp.float32)
        u = jnp.maximum(u + b1_ref[...], 0.0)
        v = jnp.dot(u, W2_ref[...], preferred_element_type=jnp.float32)
        v = jnp.maximum(v + b2_ref[...], 0.0)
        o_ref[...] = jnp.dot(v, fcW_ref[...],
                             preferred_element_type=jnp.float32) + fcb_ref[...]

    return pl.pallas_call(
        body,
        grid=(2,),
        in_specs=[
            pl.BlockSpec((N_NODES // 2, HID), lambda i: (i, 0)),
            pl.BlockSpec((N_NODES // 2, HID), lambda i: (i, 0)),
            pl.BlockSpec((N_NODES // 2, HID), lambda i: (i, 0)),
            pl.BlockSpec((HID, 2 * HID), lambda i: (0, 0)),
            pl.BlockSpec((1, 2 * HID), lambda i: (0, 0)),
            pl.BlockSpec((2 * HID, HID), lambda i: (0, 0)),
            pl.BlockSpec((1, HID), lambda i: (0, 0)),
            pl.BlockSpec((HID, NUM_CLASSES), lambda i: (0, 0)),
            pl.BlockSpec((1, NUM_CLASSES), lambda i: (0, 0)),
        ],
        out_specs=pl.BlockSpec((N_NODES // 2, NUM_CLASSES), lambda i: (i, 0)),
        out_shape=jax.ShapeDtypeStruct((N_NODES, NUM_CLASSES), jnp.float32),
    )(h1, a0, a1, W1, b1, W2, b2, fcW, fcb)


def kernel(x, edge_index, c1W1, c1b1, c1W2, c1b2,
           c2W1, c2b1, c2W2, c2b2, fcW, fcb):
    ei = edge_index.astype(jnp.int32)
    src3 = ei[0].reshape(NW, NCHUNK, CHUNK)
    dst3 = ei[1].reshape(NW, NCHUNK, CHUNK)
    zeros32 = jnp.zeros((RPT, 2 * HID), jnp.float32)
    zeros16 = jnp.zeros((RPT, HID), jnp.float32)

    p = _proj_call(x, c1W1)
    a0, a1 = _scatter_add_call(p, src3, dst3, zeros32, 2 * HID)
    h1 = _mlp1_call(p, a0, a1, c1b1.reshape(1, -1), c1W2,
                    c1b2.reshape(1, -1))
    g0, g1 = _scatter_add_call(h1, src3, dst3, zeros16, HID)
    logits = _mlp2_call(h1, g0, g1, c2W1, c2b1.reshape(1, -1), c2W2,
                        c2b2.reshape(1, -1), fcW, fcb.reshape(1, -1))
    return logits
